# Initial kernel scaffold; baseline (speedup 1.0000x reference)
import jax
import jax.numpy as jnp
from jax.experimental import pallas as pl


def kernel(x, edge_index, W1l, b1l, W1r, W2l, b2l, W2r):
    raise NotImplementedError("write your pallas kernel here")


# speedup vs baseline: 17.6258x; 17.6258x over previous
import functools

import jax
import jax.numpy as jnp
from jax import lax
from jax.experimental import pallas as pl
from jax.experimental.pallas import tpu as pltpu
from jax.experimental.pallas import tpu_sc as plsc

NC = 2
NS = 16
NW = NC * NS


def _sc_segment_sum(F, Npad, RT, W, with_deg):
    SUB = Npad // NS
    mesh = plsc.VectorSubcoreMesh(core_axis_name="c", subcore_axis_name="s",
                                  num_cores=NC, num_subcores=NS)

    def body(table, srci, dsti, *rest):
        if with_deg:
            (out, degout, tsp, acc, dacc, srcv, dstv, rows, onesv, zb,
             sem) = rest
        else:
            (out, tsp, acc, srcv, dstv, rows, zb, sem) = rest
        cid = lax.axis_index("c")
        sid = lax.axis_index("s")
        wid = sid * NC + cid

        pltpu.sync_copy(table.at[pl.ds(sid * SUB, SUB)],
                        tsp.at[pl.ds(sid * SUB, SUB)])

        zero16 = jnp.zeros((16,), jnp.float32)

        def zrow(i, carry):
            for k in range(F // 16):
                zb[i, pl.ds(k * 16, 16)] = zero16
            return carry
        lax.fori_loop(0, SUB, zrow, 0)

        if with_deg:
            one16 = jnp.ones((16,), jnp.float32)

            def orow(i, carry):
                onesv[i, :] = one16
                return carry
            lax.fori_loop(0, W, orow, 0)

        pltpu.sync_copy(zb, acc.at[pl.ds(sid * SUB, SUB)])
        if with_deg:
            pltpu.sync_copy(zb, dacc.at[pl.ds(sid * SUB, SUB)])
        plsc.subcore_barrier()

        pltpu.sync_copy(srci.at[pl.ds(wid * RT, RT)], srcv)
        pltpu.sync_copy(dsti.at[pl.ds(wid * RT, RT)], dstv)

        def step(j, carry):
            cp = pltpu.make_async_copy(tsp.at[srcv.at[j]], rows, sem)
            cp.start()
            cp.wait()
            pltpu.sync_copy(rows, acc.at[dstv.at[j]], add=True)
            if with_deg:
                pltpu.sync_copy(onesv, dacc.at[dstv.at[j]], add=True)
            return carry
        lax.fori_loop(0, RT, step, 0)
        plsc.subcore_barrier()

        pltpu.sync_copy(acc.at[pl.ds(sid * SUB, SUB)], zb)
        pltpu.sync_copy(zb, out.at[cid, pl.ds(sid * SUB, SUB)])
        if with_deg:
            pltpu.sync_copy(dacc.at[pl.ds(sid * SUB, SUB)], zb)
            pltpu.sync_copy(zb, degout.at[cid, pl.ds(sid * SUB, SUB)])

    out_type = [jax.ShapeDtypeStruct((NC, Npad, F), jnp.float32)]
    scratch = [pltpu.VMEM_SHARED((Npad, F), jnp.float32),
               pltpu.VMEM_SHARED((Npad, F), jnp.float32)]
    if with_deg:
        out_type.append(jax.ShapeDtypeStruct((NC, Npad, 16), jnp.float32))
        scratch.append(pltpu.VMEM_SHARED((Npad, 16), jnp.float32))
    scratch += [
        pltpu.VMEM((RT, W), jnp.int32),
        pltpu.VMEM((RT, W), jnp.int32),
        pltpu.VMEM((W, F), jnp.float32),
    ]
    if with_deg:
        scratch.append(pltpu.VMEM((W, 16), jnp.float32))
    scratch += [
        pltpu.VMEM((SUB, F), jnp.float32),
        pltpu.SemaphoreType.DMA,
    ]
    return pl.kernel(body, out_type=tuple(out_type), mesh=mesh,
                     scratch_types=tuple(scratch),
                     compiler_params=pltpu.CompilerParams(
                         use_tc_tiling_on_sc=False))


def _tc_lin(x, wl, wr):
    n = x.shape[0]
    h = wl.shape[0]

    def body(x_ref, wl_ref, wr_ref, xl_ref, xr_ref):
        xv = x_ref[...]
        dn = (((1,), (1,)), ((), ()))
        xl_ref[...] = lax.dot_general(xv, wl_ref[...], dn,
                                      preferred_element_type=jnp.float32)
        xr_ref[...] = lax.dot_general(xv, wr_ref[...], dn,
                                      preferred_element_type=jnp.float32)

    return pl.pallas_call(
        body,
        out_shape=[jax.ShapeDtypeStruct((n, h), jnp.float32),
                   jax.ShapeDtypeStruct((n, h), jnp.float32)],
    )(x, wl, wr)


def _tc_mid(p, dp, xr, b, wl, wr):
    n = xr.shape[0]
    c = wl.shape[0]

    def body(p_ref, d_ref, xr_ref, b_ref, wl_ref, wr_ref, hl_ref, hr_ref):
        deg = jnp.maximum(d_ref[0] + d_ref[1], 1.0)
        h = (p_ref[0] + p_ref[1]) / deg + b_ref[...] + xr_ref[...]
        h = jnp.maximum(h, 0.0)
        dn = (((1,), (1,)), ((), ()))
        hl_ref[...] = lax.dot_general(h, wl_ref[...], dn,
                                      preferred_element_type=jnp.float32)
        hr_ref[...] = lax.dot_general(h, wr_ref[...], dn,
                                      preferred_element_type=jnp.float32)

    return pl.pallas_call(
        body,
        out_shape=[jax.ShapeDtypeStruct((n, c), jnp.float32),
                   jax.ShapeDtypeStruct((n, c), jnp.float32)],
    )(p, dp, xr, b, wl, wr)


def _tc_final(q, dp, hr, b):
    n, c = hr.shape

    def body(q_ref, d_ref, hr_ref, b_ref, o_ref):
        deg16 = jnp.maximum(d_ref[0] + d_ref[1], 1.0)
        deg = jnp.concatenate([deg16] * (c // 16), axis=1)
        z = (q_ref[0] + q_ref[1]) / deg + b_ref[...] + hr_ref[...]
        m = jnp.max(z, axis=1, keepdims=True)
        e = jnp.exp(z - m)
        s = jnp.sum(e, axis=1, keepdims=True)
        o_ref[...] = z - m - jnp.log(s)

    return pl.pallas_call(
        body,
        out_shape=jax.ShapeDtypeStruct((n, c), jnp.float32),
    )(q, dp, hr, b)


def kernel(x, edge_index, W1l, b1l, W1r, W2l, b2l, W2r):
    N, D = x.shape
    E = edge_index.shape[1]
    H = W1l.shape[0]
    C = W2l.shape[0]

    Npad = -(-N // (NS * 8)) * (NS * 8)
    W = 128
    RT = -(-E // (NW * W * 8)) * 8
    Epad = NW * RT * W
    npad_e = Epad - E
    pad_idx = (N + jnp.arange(npad_e, dtype=jnp.int32) % (Npad - N))
    src1d = jnp.concatenate([edge_index[0], pad_idx])
    dst1d = jnp.concatenate([edge_index[1], pad_idx])
    src2d = src1d.reshape(NW * RT, W)
    dst2d = dst1d.reshape(NW * RT, W)

    xpad = jnp.zeros((Npad, D), jnp.float32).at[:N].set(x)

    xl, xr = _tc_lin(xpad, W1l, W1r)
    p1, dp = _sc_segment_sum(H, Npad, RT, W, True)(xl, src2d, dst2d)
    hl, hr = _tc_mid(p1, dp, xr, b1l.reshape(1, H), W2l, W2r)
    (q,) = _sc_segment_sum(C, Npad, RT, W, False)(hl, src2d, dst2d)
    out = _tc_final(q, dp, hr, b2l.reshape(1, C))
    return out[:N]


# speedup vs baseline: 21.4478x; 1.2168x over previous
import jax
import jax.numpy as jnp
from jax import lax
from jax.experimental import pallas as pl
from jax.experimental.pallas import tpu as pltpu
from jax.experimental.pallas import tpu_sc as plsc

NC = 2
NS = 16
NW = NC * NS
W = 128


def _sc_segment_sum(n_tables, Npad, ER, with_deg):
    SUB = Npad // NS
    base_rows = ER // NW
    extra = ER - base_rows * NW
    mesh = plsc.VectorSubcoreMesh(core_axis_name="c", subcore_axis_name="s",
                                  num_cores=NC, num_subcores=NS)
    MAXR = base_rows + (1 if extra else 0)

    def body(*refs):
        it = iter(refs)
        tables = [next(it) for _ in range(n_tables)]
        srci = next(it)
        dsti = next(it)
        outs = [next(it) for _ in range(n_tables)]
        degout = next(it) if with_deg else None
        tsps = [next(it) for _ in range(n_tables)]
        accs = [next(it) for _ in range(n_tables)]
        dacc = next(it) if with_deg else None
        srcv = next(it)
        dstv = next(it)
        rows = [next(it) for _ in range(n_tables)]
        onesv = next(it) if with_deg else None
        zb = next(it)
        sems = [next(it) for _ in range(n_tables)]

        cid = lax.axis_index("c")
        sid = lax.axis_index("s")
        wid = sid * NC + cid

        for t in range(n_tables):
            pltpu.sync_copy(tables[t].at[pl.ds(sid * SUB, SUB)],
                            tsps[t].at[pl.ds(sid * SUB, SUB)])

        zero16 = jnp.zeros((16,), jnp.float32)

        def zrow(i, carry):
            zb[i, :] = zero16
            return carry
        lax.fori_loop(0, SUB, zrow, 0)

        if with_deg:
            one16 = jnp.ones((16,), jnp.float32)

            def orow(i, carry):
                onesv[i, :] = one16
                return carry
            lax.fori_loop(0, W, orow, 0)

        for t in range(n_tables):
            pltpu.sync_copy(zb, accs[t].at[pl.ds(sid * SUB, SUB)])
        if with_deg:
            pltpu.sync_copy(zb, dacc.at[pl.ds(sid * SUB, SUB)])
        plsc.subcore_barrier()

        nrows = base_rows + jnp.where(wid < extra, 1, 0)
        r0 = wid * base_rows + jnp.minimum(wid, extra)
        pltpu.sync_copy(srci.at[pl.ds(r0, base_rows)],
                        srcv.at[pl.ds(0, base_rows)])
        pltpu.sync_copy(dsti.at[pl.ds(r0, base_rows)],
                        dstv.at[pl.ds(0, base_rows)])
        if extra:
            @pl.when(wid < extra)
            def _():
                pltpu.sync_copy(srci.at[pl.ds(r0 + base_rows, 1)],
                                srcv.at[pl.ds(base_rows, 1)])
                pltpu.sync_copy(dsti.at[pl.ds(r0 + base_rows, 1)],
                                dstv.at[pl.ds(base_rows, 1)])

        def step(j, carry):
            cps = [pltpu.make_async_copy(tsps[t].at[srcv.at[j]], rows[t],
                                         sems[t]) for t in range(n_tables)]
            for cp in cps:
                cp.start()
            for t in range(n_tables):
                cps[t].wait()
                pltpu.sync_copy(rows[t], accs[t].at[dstv.at[j]], add=True)
            if with_deg:
                pltpu.sync_copy(onesv, dacc.at[dstv.at[j]], add=True)
            return carry
        lax.fori_loop(0, nrows, step, 0)
        plsc.subcore_barrier()

        for t in range(n_tables):
            pltpu.sync_copy(accs[t].at[pl.ds(sid * SUB, SUB)], zb)
            pltpu.sync_copy(zb, outs[t].at[cid, pl.ds(sid * SUB, SUB)])
        if with_deg:
            pltpu.sync_copy(dacc.at[pl.ds(sid * SUB, SUB)], zb)
            pltpu.sync_copy(zb, degout.at[cid, pl.ds(sid * SUB, SUB)])

    out_type = [jax.ShapeDtypeStruct((NC, Npad, 16), jnp.float32)
                for _ in range(n_tables)]
    if with_deg:
        out_type.append(jax.ShapeDtypeStruct((NC, Npad, 16), jnp.float32))
    scratch = [pltpu.VMEM_SHARED((Npad, 16), jnp.float32)
               for _ in range(2 * n_tables)]
    if with_deg:
        scratch.append(pltpu.VMEM_SHARED((Npad, 16), jnp.float32))
    scratch += [
        pltpu.VMEM((MAXR, W), jnp.int32),
        pltpu.VMEM((MAXR, W), jnp.int32),
    ]
    scratch += [pltpu.VMEM((W, 16), jnp.float32) for _ in range(n_tables)]
    if with_deg:
        scratch.append(pltpu.VMEM((W, 16), jnp.float32))
    scratch.append(pltpu.VMEM((SUB, 16), jnp.float32))
    scratch += [pltpu.SemaphoreType.DMA for _ in range(n_tables)]
    return pl.kernel(body, out_type=tuple(out_type), mesh=mesh,
                     scratch_types=tuple(scratch),
                     compiler_params=pltpu.CompilerParams(
                         use_tc_tiling_on_sc=False))


_DN = (((1,), (0,)), ((), ()))


def _tc_lin(xa, a1, a2):
    m = xa.shape[0]

    def body(x_ref, a1_ref, a2_ref, xl_ref, xr_ref):
        xv = x_ref[...]
        xl_ref[...] = lax.dot_general(xv, a1_ref[...], _DN,
                                      preferred_element_type=jnp.float32)
        xr_ref[...] = lax.dot_general(xv, a2_ref[...], _DN,
                                      preferred_element_type=jnp.float32)

    return pl.pallas_call(
        body,
        out_shape=[jax.ShapeDtypeStruct((m, 128), jnp.float32)] * 2,
    )(xa, a1, a2)


def _tc_mid(p, dp, xrp, b1, b_mats):
    m = xrp.shape[0]

    def body(p_ref, d_ref, xr_ref, b_ref, b1m, b2m, b3m, b4m,
             hlo_ref, hhi_ref, rlo_ref, rhi_ref):
        deg = jnp.maximum(d_ref[0] + d_ref[1], 1.0)
        h = (p_ref[0] + p_ref[1]) / deg + b_ref[...] + xr_ref[...]
        h = jnp.maximum(h, 0.0)
        hlo_ref[...] = lax.dot_general(h, b1m[...], _DN,
                                       preferred_element_type=jnp.float32)
        hhi_ref[...] = lax.dot_general(h, b2m[...], _DN,
                                       preferred_element_type=jnp.float32)
        rlo_ref[...] = lax.dot_general(h, b3m[...], _DN,
                                       preferred_element_type=jnp.float32)
        rhi_ref[...] = lax.dot_general(h, b4m[...], _DN,
                                       preferred_element_type=jnp.float32)

    return pl.pallas_call(
        body,
        out_shape=[jax.ShapeDtypeStruct((m, 128), jnp.float32)] * 4,
    )(p, dp, xrp, b1, *b_mats)


def _tc_final(qlo, qhi, dp, rlo, rhi, b2lo, b2hi, ss):
    m = rlo.shape[0]

    def body(qlo_ref, qhi_ref, d_ref, rlo_ref, rhi_ref, blo_ref, bhi_ref,
             ss_ref, olo_ref, ohi_ref):
        deg = jnp.maximum(d_ref[0] + d_ref[1], 1.0)
        zlo = (qlo_ref[0] + qlo_ref[1]) / deg + blo_ref[...] + rlo_ref[...]
        zhi = (qhi_ref[0] + qhi_ref[1]) / deg + bhi_ref[...] + rhi_ref[...]
        mx = jnp.max(jnp.maximum(zlo, zhi), axis=1, keepdims=True)
        elo = jnp.exp(zlo - mx)
        ehi = jnp.exp(zhi - mx)
        s = lax.dot_general(elo + ehi, ss_ref[...], _DN,
                            preferred_element_type=jnp.float32)
        lse = mx + jnp.log(s)
        olo_ref[...] = zlo - lse
        ohi_ref[...] = zhi - lse

    return pl.pallas_call(
        body,
        out_shape=[jax.ShapeDtypeStruct((m, 128), jnp.float32)] * 2,
    )(qlo, qhi, dp, rlo, rhi, b2lo, b2hi, ss)


def kernel(x, edge_index, W1l, b1l, W1r, W2l, b2l, W2r):
    N, D = x.shape
    E = edge_index.shape[1]
    H = W1l.shape[0]
    C = W2l.shape[0]

    Npad = -(-N // 128) * 128
    M = Npad // 8
    assert E % W == 0
    ER = E // W
    src2d = edge_index[0].reshape(ER, W)
    dst2d = edge_index[1].reshape(ER, W)

    eye8 = jnp.eye(8, dtype=jnp.float32)
    a1 = jnp.kron(eye8, W1l.T)
    a2 = jnp.kron(eye8, W1r.T)
    b_mats = [jnp.kron(eye8, W2l.T[:, :16]),
              jnp.kron(eye8, W2l.T[:, 16:]),
              jnp.kron(eye8, W2r.T[:, :16]),
              jnp.kron(eye8, W2r.T[:, 16:])]
    ss = jnp.kron(eye8, jnp.ones((16, 16), jnp.float32))
    b1_128 = jnp.tile(b1l, 8).reshape(1, 128)
    b2lo = jnp.tile(b2l[:16], 8).reshape(1, 128)
    b2hi = jnp.tile(b2l[16:], 8).reshape(1, 128)

    xa = jnp.pad(x, ((0, Npad - N), (0, 0))).reshape(M, 8 * D)

    xlp, xrp = _tc_lin(xa, a1, a2)
    p1, dp = _sc_segment_sum(1, Npad, ER, True)(
        xlp.reshape(Npad, 16), src2d, dst2d)
    hlo, hhi, rlo, rhi = _tc_mid(p1.reshape(NC, M, 128),
                                 dp.reshape(NC, M, 128), xrp, b1_128, b_mats)
    qlo, qhi = _sc_segment_sum(2, Npad, ER, False)(
        hlo.reshape(Npad, 16), hhi.reshape(Npad, 16), src2d, dst2d)
    olo, ohi = _tc_final(qlo.reshape(NC, M, 128), qhi.reshape(NC, M, 128),
                         dp.reshape(NC, M, 128), rlo, rhi, b2lo, b2hi, ss)
    out = jnp.concatenate([olo.reshape(Npad, 16), ohi.reshape(Npad, 16)],
                          axis=1)
    return out[:N]
